# baseline (speedup 1.0000x reference)
import functools

import jax
import jax.numpy as jnp
from jax import lax
from jax.experimental import pallas as pl
from jax.experimental.pallas import tpu as pltpu
from jax.experimental.pallas import tpu_sc as plsc

BATCH = 16384
HIST = 50
WIDTH = 32
GRP = HIST + 1

NC = 2
NS = 16
NW = NC * NS
TBW = 4
NU = GRP * TBW
KS = 4
AHEAD = 3


def _sc_embed(crT3, vp2d, card_table, vp_table):
    mesh = plsc.VectorSubcoreMesh(core_axis_name="c", subcore_axis_name="s")

    @functools.partial(
        pl.kernel,
        mesh=mesh,
        compiler_params=pltpu.CompilerParams(
            use_tc_tiling_on_sc=False, needs_layout_passes=False),
        out_type=jax.ShapeDtypeStruct((GRP, 4, 128, 8, 128), jnp.float32),
        scratch_types=[
            pltpu.VMEM((HIST, TBW, 128), jnp.int32),
            pltpu.VMEM((TBW, 128), jnp.int32),
            pltpu.VMEM((KS, 128, WIDTH), jnp.float32),
            pltpu.VMEM((2, 4, 1, 1, 1, 8, 128), jnp.float32),
            pltpu.SemaphoreType.DMA((KS,)),
            pltpu.SemaphoreType.DMA((2,)),
        ],
    )
    def k(cr_hbm, vp_hbm, ctab_hbm, vtab_hbm, out_hbm,
          idx_v, vpi_v, sg_v, t_v, gsem, wsem):
        wid = lax.axis_index("s") * NC + lax.axis_index("c")
        pltpu.sync_copy(cr_hbm.at[:, pl.ds(wid * TBW, TBW)], idx_v)
        pltpu.sync_copy(vp_hbm.at[pl.ds(wid * TBW, TBW)], vpi_v)
        biota = lax.iota(jnp.int32, 16)
        rvecs = [biota + (j * 16) for j in range(8)]

        def fire(u, slot):
            h = u // TBW
            tb = u % TBW

            @pl.when(h < HIST)
            def _():
                hc = lax.min(h, HIST - 1)
                for q in range(4):
                    pltpu.async_copy(
                        ctab_hbm.at[idx_v.at[hc, tb, pl.ds(q * 32, 32)]],
                        sg_v.at[slot, pl.ds(q * 32, 32)], gsem.at[slot])

            @pl.when(h == HIST)
            def _():
                for q in range(4):
                    pltpu.async_copy(
                        vtab_hbm.at[vpi_v.at[tb, pl.ds(q * 32, 32)]],
                        sg_v.at[slot, pl.ds(q * 32, 32)], gsem.at[slot])

        def drain(u, slot):
            h = u // TBW
            tb = u % TBW
            hc = lax.min(h, HIST - 1)
            for q in range(4):
                pltpu.make_async_copy(
                    ctab_hbm.at[idx_v.at[hc, tb, pl.ds(q * 32, 32)]],
                    sg_v.at[slot, pl.ds(q * 32, 32)], gsem.at[slot]).wait()

        for p in range(AHEAD):
            fire(p, p)

        def unit(u, carry):
            slot = u % KS
            tslot = u % 2
            h = u // TBW
            tb = u % TBW
            drain(u, slot)
            src = sg_v.at[slot]

            def do_tile(ts):
                @pl.when(u >= 2)
                def _():
                    for tc in range(4):
                        pltpu.make_async_copy(
                            t_v.at[ts, tc],
                            out_hbm.at[pl.ds(0, 1), pl.ds(0, 1), pl.ds(0, 1)],
                            wsem.at[ts]).wait()
                for tc in range(4):
                    for cm in range(8):
                        c = tc * 8 + cm
                        cvec = jnp.full((16,), c, jnp.int32)
                        for j in range(8):
                            g = plsc.load_gather(src, [rvecs[j], cvec])
                            t_v[ts, tc, 0, 0, 0, cm, pl.ds(j * 16, 16)] = g
                for tc in range(4):
                    pltpu.async_copy(
                        t_v.at[ts, tc],
                        out_hbm.at[pl.ds(h, 1), pl.ds(tc, 1),
                                   pl.ds(wid * TBW + tb, 1)], wsem.at[ts])

            @pl.when(tslot == 0)
            def _():
                do_tile(0)

            @pl.when(tslot == 1)
            def _():
                do_tile(1)

            @pl.when(u + AHEAD < NU)
            def _():
                fire(u + AHEAD, (u + AHEAD) % KS)
            return carry

        lax.fori_loop(0, NU, unit, 0)
        for ts in range(2):
            for tc in range(4):
                pltpu.make_async_copy(
                    t_v.at[ts, tc],
                    out_hbm.at[pl.ds(0, 1), pl.ds(0, 1), pl.ds(0, 1)],
                    wsem.at[ts]).wait()

    return k(crT3, vp2d, card_table, vp_table)


def kernel(observation, card_rewards, vp_rewards, cards, card_table, vp_table):
    del observation, cards
    crT3 = card_rewards.astype(jnp.int32).T.reshape(HIST, 128, 128)
    vp2d = vp_rewards.astype(jnp.int32).reshape(128, 128)
    x = _sc_embed(crT3, vp2d, card_table, vp_table)
    return x.transpose(2, 4, 0, 1, 3).reshape(BATCH, GRP, WIDTH)


# speedup vs baseline: 1.2080x; 1.2080x over previous
import functools

import jax
import jax.numpy as jnp
from jax import lax
from jax.experimental import pallas as pl
from jax.experimental.pallas import tpu as pltpu
from jax.experimental.pallas import tpu_sc as plsc

BATCH = 16384
HIST = 50
WIDTH = 32
GRP = HIST + 1

NC = 2
NS = 16
NW = NC * NS
TBW = 4
NU = GRP * TBW
KS = 4
AHEAD = 3


def _sc_embed(crT3, vp2d, card_table, vp_table):
    mesh = plsc.VectorSubcoreMesh(core_axis_name="c", subcore_axis_name="s")

    @functools.partial(
        pl.kernel,
        mesh=mesh,
        compiler_params=pltpu.CompilerParams(
            use_tc_tiling_on_sc=False, needs_layout_passes=False),
        out_type=jax.ShapeDtypeStruct((GRP, 4, 128, 8, 128), jnp.float32),
        scratch_types=[
            pltpu.VMEM((HIST, TBW, 128), jnp.int32),
            pltpu.VMEM((TBW, 128), jnp.int32),
            pltpu.VMEM((KS, 128, WIDTH), jnp.float32),
            pltpu.VMEM((2, 4, 1, 1, 1, 8, 128), jnp.float32),
            pltpu.SemaphoreType.DMA((KS,)),
            pltpu.SemaphoreType.DMA((2,)),
        ],
    )
    def k(cr_hbm, vp_hbm, ctab_hbm, vtab_hbm, out_hbm,
          idx_v, vpi_v, sg_v, t_v, gsem, wsem):
        wid = lax.axis_index("s") * NC + lax.axis_index("c")
        pltpu.sync_copy(cr_hbm.at[:, pl.ds(wid * TBW, TBW)], idx_v)
        pltpu.sync_copy(vp_hbm.at[pl.ds(wid * TBW, TBW)], vpi_v)

        def fire(u, slot):
            h = u // TBW
            tb = u % TBW

            @pl.when(h < HIST)
            def _():
                hc = lax.min(h, HIST - 1)
                for q in range(4):
                    pltpu.async_copy(
                        ctab_hbm.at[idx_v.at[hc, tb, pl.ds(q * 32, 32)]],
                        sg_v.at[slot, pl.ds(q * 32, 32)], gsem.at[slot])

            @pl.when(h == HIST)
            def _():
                for q in range(4):
                    pltpu.async_copy(
                        vtab_hbm.at[vpi_v.at[tb, pl.ds(q * 32, 32)]],
                        sg_v.at[slot, pl.ds(q * 32, 32)], gsem.at[slot])

        def drain(u, slot):
            h = u // TBW
            tb = u % TBW
            hc = lax.min(h, HIST - 1)
            for q in range(4):
                pltpu.make_async_copy(
                    ctab_hbm.at[idx_v.at[hc, tb, pl.ds(q * 32, 32)]],
                    sg_v.at[slot, pl.ds(q * 32, 32)], gsem.at[slot]).wait()

        for p in range(AHEAD):
            fire(p, p)

        def unit(u, carry):
            slot = u % KS
            tslot = u % 2
            h = u // TBW
            tb = u % TBW
            drain(u, slot)
            src = sg_v.at[slot]
            biota = lax.iota(jnp.int32, 16)
            rvecs = [biota + (j * 16) for j in range(8)]

            def do_tile(ts):
                @pl.when(u >= 2)
                def _():
                    for tc in range(4):
                        pltpu.make_async_copy(
                            t_v.at[ts, tc],
                            out_hbm.at[pl.ds(0, 1), pl.ds(0, 1), pl.ds(0, 1)],
                            wsem.at[ts]).wait()
                for tc in range(4):
                    for cm2 in range(4):
                        ca, cb = tc * 8 + cm2 * 2, tc * 8 + cm2 * 2 + 1
                        cva = jnp.full((16,), ca, jnp.int32)
                        cvb = jnp.full((16,), cb, jnp.int32)
                        gs = ([plsc.load_gather(src, [rvecs[j], cva])
                               for j in range(8)]
                              + [plsc.load_gather(src, [rvecs[j], cvb])
                                 for j in range(8)])
                        for j in range(8):
                            t_v[ts, tc, 0, 0, 0, cm2 * 2,
                                pl.ds(j * 16, 16)] = gs[j]
                        for j in range(8):
                            t_v[ts, tc, 0, 0, 0, cm2 * 2 + 1,
                                pl.ds(j * 16, 16)] = gs[8 + j]
                for tc in range(4):
                    pltpu.async_copy(
                        t_v.at[ts, tc],
                        out_hbm.at[pl.ds(h, 1), pl.ds(tc, 1),
                                   pl.ds(wid * TBW + tb, 1)], wsem.at[ts])

            @pl.when(tslot == 0)
            def _():
                do_tile(0)

            @pl.when(tslot == 1)
            def _():
                do_tile(1)

            @pl.when(u + AHEAD < NU)
            def _():
                fire(u + AHEAD, (u + AHEAD) % KS)
            return carry

        lax.fori_loop(0, NU, unit, 0)
        for ts in range(2):
            for tc in range(4):
                pltpu.make_async_copy(
                    t_v.at[ts, tc],
                    out_hbm.at[pl.ds(0, 1), pl.ds(0, 1), pl.ds(0, 1)],
                    wsem.at[ts]).wait()

    return k(crT3, vp2d, card_table, vp_table)


def kernel(observation, card_rewards, vp_rewards, cards, card_table, vp_table):
    del observation, cards
    crT3 = card_rewards.astype(jnp.int32).T.reshape(HIST, 128, 128)
    vp2d = vp_rewards.astype(jnp.int32).reshape(128, 128)
    x = _sc_embed(crT3, vp2d, card_table, vp_table)
    return x.transpose(2, 4, 0, 1, 3).reshape(BATCH, GRP, WIDTH)


# speedup vs baseline: 1.2185x; 1.0087x over previous
import functools

import jax
import jax.numpy as jnp
from jax import lax
from jax.experimental import pallas as pl
from jax.experimental.pallas import tpu as pltpu
from jax.experimental.pallas import tpu_sc as plsc

BATCH = 16384
HIST = 50
WIDTH = 32
GRP = HIST + 1

NC = 2
NS = 16
NW = NC * NS
TBW = 4
NU = GRP * TBW
KS = 4
AHEAD = 3


def _sc_embed(crT3, vp2d, card_table, vp_table):
    mesh = plsc.VectorSubcoreMesh(core_axis_name="c", subcore_axis_name="s")

    @functools.partial(
        pl.kernel,
        mesh=mesh,
        compiler_params=pltpu.CompilerParams(
            use_tc_tiling_on_sc=False, needs_layout_passes=False),
        out_type=jax.ShapeDtypeStruct((GRP, 4, 128, 8, 128), jnp.float32),
        scratch_types=[
            pltpu.VMEM((HIST, TBW, 128), jnp.int32),
            pltpu.VMEM((TBW, 128), jnp.int32),
            pltpu.VMEM((KS, 128, 128), jnp.float32),
            pltpu.VMEM((2, 4, 1, 1, 1, 8, 128), jnp.float32),
            pltpu.SemaphoreType.DMA((KS,)),
            pltpu.SemaphoreType.DMA((2,)),
        ],
    )
    def k(cr_hbm, vp_hbm, ctab_hbm, vtab_hbm, out_hbm,
          idx_v, vpi_v, sg_v, t_v, gsem, wsem):
        wid = lax.axis_index("s") * NC + lax.axis_index("c")
        pltpu.sync_copy(cr_hbm.at[:, pl.ds(wid * TBW, TBW)], idx_v)
        pltpu.sync_copy(vp_hbm.at[pl.ds(wid * TBW, TBW)], vpi_v)

        def fire(u, slot):
            h = u // TBW
            tb = u % TBW

            @pl.when(h < HIST)
            def _():
                hc = lax.min(h, HIST - 1)
                for q in range(4):
                    pltpu.async_copy(
                        ctab_hbm.at[idx_v.at[hc, tb, pl.ds(q * 32, 32)]],
                        sg_v.at[slot, pl.ds(q * 32, 32)], gsem.at[slot])

            @pl.when(h == HIST)
            def _():
                for q in range(4):
                    pltpu.async_copy(
                        vtab_hbm.at[vpi_v.at[tb, pl.ds(q * 32, 32)]],
                        sg_v.at[slot, pl.ds(q * 32, 32)], gsem.at[slot])

        def drain(u, slot):
            h = u // TBW
            tb = u % TBW
            hc = lax.min(h, HIST - 1)
            for q in range(4):
                pltpu.make_async_copy(
                    ctab_hbm.at[idx_v.at[hc, tb, pl.ds(q * 32, 32)]],
                    sg_v.at[slot, pl.ds(q * 32, 32)], gsem.at[slot]).wait()

        for p in range(AHEAD):
            fire(p, p)

        def unit(u, carry):
            slot = u % KS
            tslot = u % 2
            h = u // TBW
            tb = u % TBW
            drain(u, slot)
            src = sg_v.at[slot]
            biota = lax.iota(jnp.int32, 16)
            rvecs = [biota + (j * 16) for j in range(8)]

            def do_tile(ts):
                @pl.when(u >= 2)
                def _():
                    for tc in range(4):
                        pltpu.make_async_copy(
                            t_v.at[ts, tc],
                            out_hbm.at[pl.ds(0, 1), pl.ds(0, 1), pl.ds(0, 1)],
                            wsem.at[ts]).wait()
                for tc in range(4):
                    for cm2 in range(4):
                        ca, cb = tc * 8 + cm2 * 2, tc * 8 + cm2 * 2 + 1
                        cva = jnp.full((16,), ca, jnp.int32)
                        cvb = jnp.full((16,), cb, jnp.int32)
                        gs = ([plsc.load_gather(src, [rvecs[j], cva])
                               for j in range(8)]
                              + [plsc.load_gather(src, [rvecs[j], cvb])
                                 for j in range(8)])
                        for j in range(8):
                            t_v[ts, tc, 0, 0, 0, cm2 * 2,
                                pl.ds(j * 16, 16)] = gs[j]
                        for j in range(8):
                            t_v[ts, tc, 0, 0, 0, cm2 * 2 + 1,
                                pl.ds(j * 16, 16)] = gs[8 + j]
                for tc in range(4):
                    pltpu.async_copy(
                        t_v.at[ts, tc],
                        out_hbm.at[pl.ds(h, 1), pl.ds(tc, 1),
                                   pl.ds(wid * TBW + tb, 1)], wsem.at[ts])

            @pl.when(tslot == 0)
            def _():
                do_tile(0)

            @pl.when(tslot == 1)
            def _():
                do_tile(1)

            @pl.when(u + AHEAD < NU)
            def _():
                fire(u + AHEAD, (u + AHEAD) % KS)
            return carry

        lax.fori_loop(0, NU, unit, 0)
        for ts in range(2):
            for tc in range(4):
                pltpu.make_async_copy(
                    t_v.at[ts, tc],
                    out_hbm.at[pl.ds(0, 1), pl.ds(0, 1), pl.ds(0, 1)],
                    wsem.at[ts]).wait()

    return k(crT3, vp2d, card_table, vp_table)


def kernel(observation, card_rewards, vp_rewards, cards, card_table, vp_table):
    del observation, cards
    crT3 = card_rewards.astype(jnp.int32).T.reshape(HIST, 128, 128)
    vp2d = vp_rewards.astype(jnp.int32).reshape(128, 128)
    ctp = jnp.pad(card_table, ((0, 0), (0, 128 - WIDTH)))
    vtp = jnp.pad(vp_table, ((0, 0), (0, 128 - WIDTH)))
    x = _sc_embed(crT3, vp2d, ctp, vtp)
    return x.transpose(2, 4, 0, 1, 3).reshape(BATCH, GRP, WIDTH)
